# Initial kernel scaffold; baseline (speedup 1.0000x reference)
import jax
import jax.numpy as jnp
from jax.experimental import pallas as pl


def kernel(x, edge_index, W_self0, W_nbr0, b0, gamma0, beta0, W_self1, W_nbr1, b1, gamma1, beta1, W_self2, W_nbr2, b2, gamma2, beta2):
    raise NotImplementedError("write your pallas kernel here")


# speedup vs baseline: 4.3756x; 4.3756x over previous
import functools

import jax
import jax.numpy as jnp
from jax import lax
from jax.experimental import pallas as pl
from jax.experimental.pallas import tpu as pltpu
from jax.experimental.pallas import tpu_sc as plsc

N = 10000
NP = 10240
D = 128
DW = 128
NC = 2
NS = 16
NW = NC * NS
CH = 128
CPW = 79
E_PAD = NW * CPW * CH
RPS = NP // NS

_MESH = plsc.VectorSubcoreMesh(core_axis_name="c", subcore_axis_name="s")


def _agg_body(h_hbm, src_hbm, dst_hbm, agg_out, src_v, dst_v, rows, acc, sem):
  c = lax.axis_index("c")
  s = lax.axis_index("s")
  w = s * NC + c
  z16 = jnp.zeros((16,), jnp.float32)

  def zero_rows(k, carry):
    rows[k // 8, pl.ds((k % 8) * 16, 16)] = z16
    return carry
  lax.fori_loop(0, CH * 8, zero_rows, 0)
  for k in range(RPS // CH):
    pltpu.sync_copy(rows.at[pl.ds(0, CH)],
                    acc.at[pl.ds(s * RPS + k * CH, CH)])

  pltpu.sync_copy(src_hbm.at[w], src_v)
  pltpu.sync_copy(dst_hbm.at[w], dst_v)
  plsc.subcore_barrier()

  def step(i, carry):
    pltpu.async_copy(h_hbm.at[src_v.at[i]], rows, sem).wait()
    pltpu.sync_copy(rows, acc.at[dst_v.at[i]], add=True)
    return carry
  lax.fori_loop(0, CPW, step, 0)

  plsc.subcore_barrier()
  pltpu.sync_copy(acc.at[pl.ds(s * RPS, RPS)],
                  agg_out.at[c, pl.ds(s * RPS, RPS)])


_agg = pl.kernel(
    _agg_body,
    out_type=[jax.ShapeDtypeStruct((NC, NP, D), jnp.float32)],
    mesh=_MESH,
    scratch_types=[
        pltpu.VMEM((CPW, CH), jnp.int32),
        pltpu.VMEM((CPW, CH), jnp.int32),
        pltpu.VMEM((CH, D), jnp.float32),
        pltpu.VMEM_SHARED((NP, D), jnp.float32),
        pltpu.SemaphoreType.DMA,
    ],
)


def _deg_body(dst_hbm, deg_out, dst_v, ones_v, dacc):
  c = lax.axis_index("c")
  s = lax.axis_index("s")
  w = s * NC + c
  z16 = jnp.zeros((16,), jnp.float32)
  o16 = jnp.ones((16,), jnp.float32)

  def zero_ones(k, carry):
    ones_v[k // 8, pl.ds((k % 8) * 16, 16)] = z16
    return carry
  lax.fori_loop(0, CH * (DW // 16), zero_ones, 0)
  for k in range(RPS // CH):
    pltpu.sync_copy(ones_v, dacc.at[pl.ds(s * RPS + k * CH, CH)])

  def fill_ones(k, carry):
    ones_v[k // 8, pl.ds((k % 8) * 16, 16)] = o16
    return carry
  lax.fori_loop(0, CH * (DW // 16), fill_ones, 0)

  pltpu.sync_copy(dst_hbm.at[w], dst_v)
  plsc.subcore_barrier()

  def step(i, carry):
    pltpu.sync_copy(ones_v, dacc.at[dst_v.at[i]], add=True)
    return carry
  lax.fori_loop(0, CPW, step, 0)

  plsc.subcore_barrier()
  pltpu.sync_copy(dacc.at[pl.ds(s * RPS, RPS)],
                  deg_out.at[c, pl.ds(s * RPS, RPS)])


_deg = pl.kernel(
    _deg_body,
    out_type=[jax.ShapeDtypeStruct((NC, NP, DW), jnp.float32)],
    mesh=_MESH,
    scratch_types=[
        pltpu.VMEM((CPW, CH), jnp.int32),
        pltpu.VMEM((CH, DW), jnp.float32),
        pltpu.VMEM_SHARED((NP, DW), jnp.float32),
    ],
)

BR = 2048


def _dense_body(h, pp, dp, ws, wn, bb, gg, be, out):
  deg = dp[0, :, 0] + dp[1, :, 0] + 1.0
  agg = (pp[0] + pp[1] + h[...]) / deg[:, None]
  y = (jnp.dot(h[...], ws[...], preferred_element_type=jnp.float32)
       + jnp.dot(agg, wn[...], preferred_element_type=jnp.float32)
       + bb[...])
  mu = jnp.mean(y, axis=-1, keepdims=True)
  yc = y - mu
  var = jnp.mean(yc * yc, axis=-1, keepdims=True)
  yn = yc * lax.rsqrt(var + 1e-5) * gg[...] + be[...]
  out[...] = jnp.where(yn > 0, yn, jnp.exp(jnp.minimum(yn, 0.0)) - 1.0)


_dense = pl.pallas_call(
    _dense_body,
    grid=(NP // BR,),
    in_specs=[
        pl.BlockSpec((BR, D), lambda r: (r, 0)),
        pl.BlockSpec((NC, BR, D), lambda r: (0, r, 0)),
        pl.BlockSpec((NC, BR, DW), lambda r: (0, r, 0)),
        pl.BlockSpec((D, D), lambda r: (0, 0)),
        pl.BlockSpec((D, D), lambda r: (0, 0)),
        pl.BlockSpec((1, D), lambda r: (0, 0)),
        pl.BlockSpec((1, D), lambda r: (0, 0)),
        pl.BlockSpec((1, D), lambda r: (0, 0)),
    ],
    out_specs=pl.BlockSpec((BR, D), lambda r: (r, 0)),
    out_shape=jax.ShapeDtypeStruct((NP, D), jnp.float32),
)


def kernel(x, edge_index, W_self0, W_nbr0, b0, gamma0, beta0,
           W_self1, W_nbr1, b1, gamma1, beta1,
           W_self2, W_nbr2, b2, gamma2, beta2):
  e = edge_index.shape[1]
  pad = E_PAD - e
  src = jnp.concatenate([edge_index[0], jnp.zeros((pad,), jnp.int32)])
  dst = jnp.concatenate([edge_index[1], jnp.full((pad,), N, jnp.int32)])
  srcp = src.reshape(NW, CPW, CH)
  dstp = dst.reshape(NW, CPW, CH)

  params = [(W_self0, W_nbr0, b0, gamma0, beta0),
            (W_self1, W_nbr1, b1, gamma1, beta1),
            (W_self2, W_nbr2, b2, gamma2, beta2)]

  h = jnp.pad(x, ((0, NP - N), (0, 0)))
  (dp,) = _deg(dstp)
  for ws, wn, bb, gg, be in params:
    (parts,) = _agg(h, srcp, dstp)
    h = _dense(h, parts, dp, ws, wn,
               bb.reshape(1, D), gg.reshape(1, D), be.reshape(1, D))
  return h[:N]
